# baseline (speedup 1.0000x reference)
import functools

import jax
import jax.numpy as jnp
from jax import lax
from jax.experimental import pallas as pl
from jax.experimental.pallas import tpu as pltpu
from jax.experimental.pallas import tpu_sc as plsc

N_NODES = 10000
N_FEAT = 128
N_EDGES = 320000
BN_EPS = 1e-5

NC = 2
NS = 16
NW = NC * NS

CHUNK = 120
CPT = 84
E_PAD = NW * CPT * CHUNK
N_PAD = 10112
ROWS_PER_TILE = N_PAD // NS

_sc_mesh = plsc.VectorSubcoreMesh(core_axis_name="c", subcore_axis_name="s")


@functools.partial(
    pl.kernel,
    mesh=_sc_mesh,
    out_type=jax.ShapeDtypeStruct((NC, N_PAD, N_FEAT), jnp.float32),
    scratch_types=(
        [
            pltpu.VMEM_SHARED((N_PAD, N_FEAT), jnp.float32),
            pltpu.VMEM((6, 2, CHUNK), jnp.int32),
            pltpu.VMEM((3, CHUNK, N_FEAT), jnp.float32),
        ]
        + [pltpu.SemaphoreType.DMA] * 13
    ),
)
def _sc_agg(x_hbm, sd_hbm, zeros_hbm, out_hbm,
            acc, idx, rows, *sems):
    c = lax.axis_index("c")
    s = lax.axis_index("s")
    w = s * NC + c
    sem_i = sems[0:6]
    sem_g = sems[6:9]
    sem_s = sems[9:12]
    sem_z = sems[12]

    pltpu.async_copy(zeros_hbm.at[pl.ds(s * ROWS_PER_TILE, ROWS_PER_TILE)],
                     acc.at[pl.ds(s * ROWS_PER_TILE, ROWS_PER_TILE)], sem_z)

    def idx_start(jd, kI):
        pltpu.async_copy(sd_hbm.at[w, jd], idx.at[kI], sem_i[kI])

    def idx_wait(kI):
        pltpu.make_async_copy(sd_hbm.at[w, 0], idx.at[kI], sem_i[kI]).wait()

    def gather_start(jd, kI, kR):
        pltpu.async_copy(x_hbm.at[idx.at[kI, 0]], rows.at[kR], sem_g[kR])

    def gather_wait(kR):
        pltpu.make_async_copy(x_hbm.at[pl.ds(0, CHUNK)], rows.at[kR],
                              sem_g[kR]).wait()

    def scat_start(kI, kR):
        pltpu.async_copy(rows.at[kR], acc.at[idx.at[kI, 1]], sem_s[kR],
                         add=True)

    def scat_wait(kR):
        pltpu.make_async_copy(rows.at[kR], acc.at[pl.ds(0, CHUNK)],
                              sem_s[kR]).wait()

    def step(jd, k, do_scat_wait, do_prefetch, do_drain):
        if do_scat_wait:
            scat_wait(k % 3)
        if do_prefetch:
            idx_start(jd + 3, (k + 3) % 6)
        idx_wait(k)
        gather_start(jd, k, k % 3)
        if do_drain:
            gather_wait((k - 2) % 3)
            scat_start((k - 2) % 6, (k - 2) % 3)

    for k in range(3):
        idx_start(k, k)
    for k in range(2):
        step(k, k, False, True, False)
    pltpu.make_async_copy(
        zeros_hbm.at[pl.ds(s * ROWS_PER_TILE, ROWS_PER_TILE)],
        acc.at[pl.ds(s * ROWS_PER_TILE, ROWS_PER_TILE)], sem_z).wait()
    plsc.subcore_barrier()
    for k in range(2, 6):
        step(k, k, k >= 3, True, k >= 2)

    def body(lv, _):
        jb = lv * 6
        for k in range(6):
            step(jb + k, k, True, True, True)
        return ()

    lax.fori_loop(1, CPT // 6 - 1, body, ())

    for k in range(6):
        step(CPT - 6 + k, k, True, k < 3, True)

    gather_wait((CPT - 2) % 3)
    scat_start((CPT - 2) % 6, (CPT - 2) % 3)
    gather_wait((CPT - 1) % 3)
    scat_start((CPT - 1) % 6, (CPT - 1) % 3)
    for r in range(3):
        scat_wait(r)

    plsc.subcore_barrier()
    pltpu.sync_copy(acc.at[pl.ds(s * ROWS_PER_TILE, ROWS_PER_TILE)],
                    out_hbm.at[c, pl.ds(s * ROWS_PER_TILE, ROWS_PER_TILE)])


NB = 10
RB = N_NODES // NB


def _dense_body(x_ref, p_ref, eps_ref, w_ref, b_ref, g_ref, be_ref, o_ref,
                y_scr, st_scr, *, pad_out):
    i = pl.program_id(0)
    h = ((1.0 + eps_ref[0, 0]) * x_ref[...]
         + p_ref[0, :, :] + p_ref[1, :, :])
    y = lax.dot_general(h, w_ref[...], (((1,), (1,)), ((), ())),
                        preferred_element_type=jnp.float32)
    y = y + b_ref[...]
    y_scr[pl.ds(i * RB, RB), :] = y
    ps = jnp.concatenate(
        [jnp.sum(y, axis=0, keepdims=True),
         jnp.sum(y * y, axis=0, keepdims=True)], axis=0)

    @pl.when(i == 0)
    def _():
        st_scr[0:2, :] = ps

    @pl.when(i > 0)
    def _():
        st_scr[0:2, :] = st_scr[0:2, :] + ps

    @pl.when(i == NB - 1)
    def _():
        n = jnp.float32(N_NODES)
        mean = st_scr[0:1, :] / n
        var = st_scr[1:2, :] / n - mean * mean
        z = ((y_scr[...] - mean) * lax.rsqrt(var + BN_EPS) * g_ref[...]
             + be_ref[...])
        o_ref[0:N_NODES, :] = jnp.maximum(z, 0.0)
        if pad_out:
            o_ref[N_NODES:, :] = jnp.zeros((N_PAD - N_NODES, N_FEAT),
                                           jnp.float32)


def _dense(x, p, eps, W, b, g, be, pad_out):
    rows_out = N_PAD if pad_out else N_NODES
    return pl.pallas_call(
        functools.partial(_dense_body, pad_out=pad_out),
        grid=(NB,),
        in_specs=[
            pl.BlockSpec((RB, N_FEAT), lambda i: (i, 0)),
            pl.BlockSpec((NC, RB, N_FEAT), lambda i: (0, i, 0)),
            pl.BlockSpec((1, 1), lambda i: (0, 0)),
            pl.BlockSpec((N_FEAT, N_FEAT), lambda i: (0, 0)),
            pl.BlockSpec((1, N_FEAT), lambda i: (0, 0)),
            pl.BlockSpec((1, N_FEAT), lambda i: (0, 0)),
            pl.BlockSpec((1, N_FEAT), lambda i: (0, 0)),
        ],
        out_specs=pl.BlockSpec((rows_out, N_FEAT), lambda i: (0, 0)),
        out_shape=jax.ShapeDtypeStruct((rows_out, N_FEAT), jnp.float32),
        scratch_shapes=[
            pltpu.VMEM((N_NODES, N_FEAT), jnp.float32),
            pltpu.VMEM((8, N_FEAT), jnp.float32),
        ],
    )(x, p, eps.astype(jnp.float32).reshape(1, 1), W,
      b.reshape(1, N_FEAT), g.reshape(1, N_FEAT), be.reshape(1, N_FEAT))


def kernel(x, edge_index, eps1, W1, b1, g1, be1, eps2, W2, b2, g2, be2):
    n_extra = E_PAD - N_EDGES
    pad_idx = (N_NODES
               + (jnp.arange(n_extra, dtype=jnp.int32) % (N_PAD - N_NODES)))
    src = jnp.concatenate([edge_index[0], pad_idx]).reshape(NW, CPT, 1, CHUNK)
    dst = jnp.concatenate([edge_index[1], pad_idx]).reshape(NW, CPT, 1, CHUNK)
    sd = jnp.concatenate([src, dst], axis=2)

    xp = jnp.pad(x, ((0, N_PAD - N_NODES), (0, 0)))
    zeros = jnp.zeros((N_PAD, N_FEAT), jnp.float32)

    p1 = _sc_agg(xp, sd, zeros)
    h1 = _dense(x, p1, eps1, W1, b1, g1, be1, pad_out=True)
    p2 = _sc_agg(h1, sd, zeros)
    h2 = _dense(h1, p2, eps2, W2, b2, g2, be2, pad_out=False)
    return h2


# speedup vs baseline: 1.0353x; 1.0353x over previous
import functools

import jax
import jax.numpy as jnp
from jax import lax
from jax.experimental import pallas as pl
from jax.experimental.pallas import tpu as pltpu
from jax.experimental.pallas import tpu_sc as plsc

N_NODES = 10000
N_FEAT = 128
N_EDGES = 320000
BN_EPS = 1e-5

NC = 2
NS = 16
NW = NC * NS

CHUNK = 120
CPT = 84
E_PAD = NW * CPT * CHUNK
N_PAD = 10112
ROWS_PER_TILE = N_PAD // NS

_sc_mesh = plsc.VectorSubcoreMesh(core_axis_name="c", subcore_axis_name="s")


@functools.partial(
    pl.kernel,
    mesh=_sc_mesh,
    out_type=jax.ShapeDtypeStruct((NC, N_PAD, N_FEAT), jnp.float32),
    scratch_types=(
        [
            pltpu.VMEM_SHARED((N_PAD, N_FEAT), jnp.float32),
            pltpu.VMEM((6, 2, CHUNK), jnp.int32),
            pltpu.VMEM((3, CHUNK, N_FEAT), jnp.float32),
        ]
        + [pltpu.SemaphoreType.DMA] * 13
    ),
)
def _sc_agg(x_hbm, sd_hbm, zeros_hbm, out_hbm,
            acc, idx, rows, *sems):
    c = lax.axis_index("c")
    s = lax.axis_index("s")
    w = s * NC + c
    sem_i = sems[0:6]
    sem_g = sems[6:9]
    sem_s = sems[9:12]
    sem_z = sems[12]

    pltpu.async_copy(zeros_hbm.at[pl.ds(s * ROWS_PER_TILE, ROWS_PER_TILE)],
                     acc.at[pl.ds(s * ROWS_PER_TILE, ROWS_PER_TILE)], sem_z)

    def idx_start(jd, kI):
        pltpu.async_copy(sd_hbm.at[w, jd], idx.at[kI], sem_i[kI])

    def idx_wait(kI):
        pltpu.make_async_copy(sd_hbm.at[w, 0], idx.at[kI], sem_i[kI]).wait()

    def gather_start(jd, kI, kR):
        pltpu.async_copy(x_hbm.at[idx.at[kI, 0]], rows.at[kR], sem_g[kR])

    def gather_wait(kR):
        pltpu.make_async_copy(x_hbm.at[pl.ds(0, CHUNK)], rows.at[kR],
                              sem_g[kR]).wait()

    def scat_start(kI, kR):
        pltpu.async_copy(rows.at[kR], acc.at[idx.at[kI, 1]], sem_s[kR],
                         add=True)

    def scat_wait(kR):
        pltpu.make_async_copy(rows.at[kR], acc.at[pl.ds(0, CHUNK)],
                              sem_s[kR]).wait()

    def step(jd, k, do_scat_wait, do_prefetch, do_drain):
        if do_scat_wait:
            scat_wait(k % 3)
        if do_prefetch:
            idx_start(jd + 3, (k + 3) % 6)
        idx_wait(k)
        gather_start(jd, k, k % 3)
        if do_drain:
            gather_wait((k - 2) % 3)
            scat_start((k - 2) % 6, (k - 2) % 3)

    for k in range(3):
        idx_start(k, k)
    for k in range(2):
        step(k, k, False, True, False)
    pltpu.make_async_copy(
        zeros_hbm.at[pl.ds(s * ROWS_PER_TILE, ROWS_PER_TILE)],
        acc.at[pl.ds(s * ROWS_PER_TILE, ROWS_PER_TILE)], sem_z).wait()
    plsc.subcore_barrier()
    for k in range(2, 6):
        step(k, k, k >= 3, True, k >= 2)

    def body(lv, _):
        jb = lv * 6
        for k in range(6):
            step(jb + k, k, True, True, True)
        return ()

    lax.fori_loop(1, CPT // 6 - 1, body, ())

    for k in range(6):
        step(CPT - 6 + k, k, True, k < 3, True)

    gather_wait((CPT - 2) % 3)
    scat_start((CPT - 2) % 6, (CPT - 2) % 3)
    gather_wait((CPT - 1) % 3)
    scat_start((CPT - 1) % 6, (CPT - 1) % 3)
    for r in range(3):
        scat_wait(r)

    plsc.subcore_barrier()
    pltpu.sync_copy(acc.at[pl.ds(s * ROWS_PER_TILE, ROWS_PER_TILE)],
                    out_hbm.at[c, pl.ds(s * ROWS_PER_TILE, ROWS_PER_TILE)])


def _dense_body(x_ref, p_ref, eps_ref, w_ref, b_ref, g_ref, be_ref, o_ref,
                *, pad_out):
    xs = x_ref[0:N_NODES, :]
    agg = p_ref[0, 0:N_NODES, :] + p_ref[1, 0:N_NODES, :]
    h = (1.0 + eps_ref[0, 0]) * xs + agg
    y = lax.dot_general(h, w_ref[...], (((1,), (1,)), ((), ())),
                        preferred_element_type=jnp.float32)
    y = y + b_ref[...]
    mean = jnp.mean(y, axis=0, keepdims=True)
    var = jnp.mean((y - mean) ** 2, axis=0, keepdims=True)
    z = (y - mean) * lax.rsqrt(var + BN_EPS) * g_ref[...] + be_ref[...]
    z = jnp.maximum(z, 0.0)
    o_ref[0:N_NODES, :] = z
    if pad_out:
        o_ref[N_NODES:, :] = jnp.zeros((N_PAD - N_NODES, N_FEAT), jnp.float32)


def _dense(x, p, eps, W, b, g, be, pad_out):
    rows_out = N_PAD if pad_out else N_NODES
    return pl.pallas_call(
        functools.partial(_dense_body, pad_out=pad_out),
        out_shape=jax.ShapeDtypeStruct((rows_out, N_FEAT), jnp.float32),
    )(x, p, eps.astype(jnp.float32).reshape(1, 1), W,
      b.reshape(1, N_FEAT), g.reshape(1, N_FEAT), be.reshape(1, N_FEAT))


def kernel(x, edge_index, eps1, W1, b1, g1, be1, eps2, W2, b2, g2, be2):
    n_extra = E_PAD - N_EDGES
    pad_idx = (N_NODES
               + (jnp.arange(n_extra, dtype=jnp.int32) % (N_PAD - N_NODES)))
    src = jnp.concatenate([edge_index[0], pad_idx]).reshape(NW, CPT, 1, CHUNK)
    dst = jnp.concatenate([edge_index[1], pad_idx]).reshape(NW, CPT, 1, CHUNK)
    sd = jnp.concatenate([src, dst], axis=2)

    xp = jnp.pad(x, ((0, N_PAD - N_NODES), (0, 0)))
    zeros = jnp.zeros((N_PAD, N_FEAT), jnp.float32)

    p1 = _sc_agg(xp, sd, zeros)
    h1 = _dense(xp, p1, eps1, W1, b1, g1, be1, pad_out=True)
    p2 = _sc_agg(h1, sd, zeros)
    h2 = _dense(h1, p2, eps2, W2, b2, g2, be2, pad_out=False)
    return h2
